# baseline (speedup 1.0000x reference)
import functools

import jax
import jax.numpy as jnp
from jax import lax
from jax.experimental import pallas as pl
from jax.experimental.pallas import tpu as pltpu
from jax.experimental.pallas import tpu_sc as plsc

N = 10000
E = 160000
IN_DIM = 256
H = 4
D = 128
NB_ROWS = 1000
NS = 16
NC = 2
BSZ = 80
NBT = 128
EC = NBT * BSZ
E2 = NS * EC
ROWS_PER_TILE = N // NS
ZR = 25


def _prep_body(x_ref, w_ref, al_ref, ar_ref, feat_ref, el_ref, er_ref):
    xb = x_ref[...]
    fb = jnp.dot(xb, w_ref[0], preferred_element_type=jnp.float32)
    f3 = fb.reshape(NB_ROWS, H, D)
    el_ref[0] = (f3 * al_ref[0][None]).sum(-1)
    er_ref[0] = (f3 * ar_ref[0][None]).sum(-1)
    feat_ref[0] = f3.transpose(1, 0, 2)


def _prep(X, Wst, alst, arst):
    return pl.pallas_call(
        _prep_body,
        grid=(2, N // NB_ROWS),
        in_specs=[
            pl.BlockSpec((NB_ROWS, IN_DIM), lambda v, i: (i, 0)),
            pl.BlockSpec((1, IN_DIM, H * D), lambda v, i: (v, 0, 0)),
            pl.BlockSpec((1, H, D), lambda v, i: (v, 0, 0)),
            pl.BlockSpec((1, H, D), lambda v, i: (v, 0, 0)),
        ],
        out_specs=[
            pl.BlockSpec((1, H, NB_ROWS, D), lambda v, i: (v, 0, i, 0)),
            pl.BlockSpec((1, NB_ROWS, H), lambda v, i: (v, i, 0)),
            pl.BlockSpec((1, NB_ROWS, H), lambda v, i: (v, i, 0)),
        ],
        out_shape=[
            jax.ShapeDtypeStruct((2, H, N, D), jnp.float32),
            jax.ShapeDtypeStruct((2, N, H), jnp.float32),
            jax.ShapeDtypeStruct((2, N, H), jnp.float32),
        ],
    )(X, Wst, alst, arst)


def _sck1_body(el_hbm, er_hbm, edges_hbm,
               w_hbm, den_hbm,
               el_v, er_v, src_c, dst_c, w_c, den_v):
    cid = lax.axis_index("c")
    sid = lax.axis_index("s")
    zeros16 = jnp.zeros((16,), jnp.float32)

    for v in range(2):
        for hh in range(2):
            h = cid * 2 + hh

            pltpu.sync_copy(el_hbm.at[v, h], el_v)
            pltpu.sync_copy(er_hbm.at[v, h], er_v.at[pl.ds(0, N)])
            er_v[pl.ds(N, 16)] = zeros16
            pltpu.sync_copy(edges_hbm.at[v, 0, sid], src_c)
            pltpu.sync_copy(edges_hbm.at[v, 1, sid], dst_c)

            def _zd(i, _):
                den_v[pl.ds(i * 16, 16)] = zeros16
                return 0
            lax.fori_loop(0, (N + 16) // 16, _zd, 0)

            def _wk(k, _):
                s16 = src_c[pl.ds(k * 16, 16)]
                d16 = dst_c[pl.ds(k * 16, 16)]
                e16 = (plsc.load_gather(el_v, [s16])
                       + plsc.load_gather(er_v, [d16]))
                e16 = jnp.where(e16 >= 0.0, e16, e16 * 0.2)
                w16 = jnp.exp(e16)
                w_c[pl.ds(k * 16, 16)] = w16
                plsc.addupdate_scatter(den_v, [d16], w16)
                return 0
            lax.fori_loop(0, EC // 16, _wk, 0)

            pltpu.sync_copy(w_c, w_hbm.at[v, h, sid])
            pltpu.sync_copy(den_v.at[pl.ds(0, N)], den_hbm.at[v, h, sid])


def _sck1(el_t, er_t, edges4):
    mesh = plsc.VectorSubcoreMesh(core_axis_name="c", subcore_axis_name="s")
    fn = functools.partial(
        pl.kernel,
        out_type=[
            jax.ShapeDtypeStruct((2, H, NS, EC), jnp.float32),
            jax.ShapeDtypeStruct((2, H, NS, N), jnp.float32),
        ],
        mesh=mesh,
        compiler_params=pltpu.CompilerParams(use_tc_tiling_on_sc=False,
                                             needs_layout_passes=False),
        scratch_types=[
            pltpu.VMEM((N,), jnp.float32),
            pltpu.VMEM((N + 16,), jnp.float32),
            pltpu.VMEM((EC,), jnp.int32),
            pltpu.VMEM((EC,), jnp.int32),
            pltpu.VMEM((EC,), jnp.float32),
            pltpu.VMEM((N + 16,), jnp.float32),
        ],
    )(_sck1_body)
    return fn(el_t, er_t, edges4)


def _sck2_body(feat_hbm, edges_hbm, w_hbm,
               acc_hbm,
               acc_sh,
               gb, sidx, didx, wr, gidx, zero_v,
               gsem, isem):
    cid = lax.axis_index("c")
    sid = lax.axis_index("s")
    row0 = sid * ROWS_PER_TILE
    zeros16 = jnp.zeros((16,), jnp.float32)

    def _zz(i, _):
        for j in range(D // 16):
            zero_v[i, pl.ds(j * 16, 16)] = zeros16
        return 0
    lax.fori_loop(0, ZR, _zz, 0)

    def _issue_idx(v, h, g, p):
        pltpu.async_copy(edges_hbm.at[v, 0, sid, g], sidx.at[p], isem.at[p])
        pltpu.async_copy(edges_hbm.at[v, 1, sid, g], didx.at[p], isem.at[p])
        pltpu.async_copy(w_hbm.at[v, h, sid, g], wr.at[p], isem.at[p])

    def _wait_idx(v, h, p):
        pltpu.make_async_copy(edges_hbm.at[v, 0, sid, 0], sidx.at[p],
                              isem.at[p]).wait()
        pltpu.make_async_copy(edges_hbm.at[v, 1, sid, 0], didx.at[p],
                              isem.at[p]).wait()
        pltpu.make_async_copy(w_hbm.at[v, h, sid, 0], wr.at[p],
                              isem.at[p]).wait()

    def _build_and_gather(base, p):
        def _gi(k, _):
            s16 = sidx[p, 0, pl.ds(k * 16, 16)]
            gidx[p, pl.ds(k * 16, 16)] = s16 + base
            return 0
        lax.fori_loop(0, BSZ // 16, _gi, 0)
        pltpu.async_copy(feat_hbm.at[gidx.at[p]], gb.at[p], gsem.at[p])

    def _wait_gather(p):
        pltpu.make_async_copy(feat_hbm.at[gidx.at[p]], gb.at[p],
                              gsem.at[p]).wait()

    def _scale(p):
        def _sk(k, _):
            w16 = wr[p, 0, pl.ds(k * 16, 16)]
            for i16 in range(16):
                w = w16[i16]
                i = k * 16 + i16
                for j in range(D // 16):
                    gb[p, i, pl.ds(j * 16, 16)] = (
                        gb[p, i, pl.ds(j * 16, 16)] * w)
            return 0
        lax.fori_loop(0, BSZ // 16, _sk, 0)

    for v in range(2):
        for hh in range(2):
            h = cid * 2 + hh
            base = (v * H + h) * N

            def _za(z, _):
                pltpu.sync_copy(zero_v, acc_sh.at[pl.ds(row0 + z * ZR, ZR)])
                return 0
            lax.fori_loop(0, ROWS_PER_TILE // ZR, _za, 0)

            plsc.subcore_barrier()

            _issue_idx(v, h, 0, 0)
            _issue_idx(v, h, 1, 1)
            _wait_idx(v, h, 0)
            _build_and_gather(base, 0)

            def _pair(t, _):
                for p in range(2):
                    g = 2 * t + p
                    q = 1 - p
                    @pl.when(g + 1 <= NBT - 1)
                    def _():
                        _wait_idx(v, h, q)
                        _build_and_gather(base, q)
                    _wait_gather(p)
                    _scale(p)
                    pltpu.sync_copy(gb.at[p], acc_sh.at[didx.at[p, 0]],
                                    add=True)
                    @pl.when(g + 2 <= NBT - 1)
                    def _():
                        _issue_idx(v, h, g + 2, p)
                return 0
            lax.fori_loop(0, NBT // 2, _pair, 0)

            plsc.subcore_barrier()

            def _wo(z, _):
                r = row0 + z * ZR
                pltpu.sync_copy(acc_sh.at[pl.ds(r, ZR)],
                                acc_hbm.at[v, h, pl.ds(r, ZR)])
                return 0
            lax.fori_loop(0, ROWS_PER_TILE // ZR, _wo, 0)

            plsc.subcore_barrier()


def _sck2(feat_flat, edges6, w6):
    mesh = plsc.VectorSubcoreMesh(core_axis_name="c", subcore_axis_name="s")
    fn = functools.partial(
        pl.kernel,
        out_type=jax.ShapeDtypeStruct((2, H, N, D), jnp.float32),
        mesh=mesh,
        compiler_params=pltpu.CompilerParams(use_tc_tiling_on_sc=False,
                                             needs_layout_passes=False),
        scratch_types=[
            pltpu.VMEM_SHARED((N + 8, D), jnp.float32),
            pltpu.VMEM((2, BSZ, D), jnp.float32),
            pltpu.VMEM((2, 1, BSZ), jnp.int32),
            pltpu.VMEM((2, 1, BSZ), jnp.int32),
            pltpu.VMEM((2, 1, BSZ), jnp.float32),
            pltpu.VMEM((2, BSZ), jnp.int32),
            pltpu.VMEM((ZR, D), jnp.float32),
            pltpu.SemaphoreType.DMA((2,)),
            pltpu.SemaphoreType.DMA((2,)),
        ],
    )(_sck2_body)
    return fn(feat_flat, edges6, w6)


def _comb_body(acc_ref, den_ref, b_ref, out_ref):
    dblk = den_ref[0].sum(axis=2)
    for h in range(H):
        terms = []
        for v in range(2):
            numer = acc_ref[v, h]
            dd = dblk[v, h][:, None] + 1e-9
            terms.append(numer / dd + b_ref[v, h][None])
        out_ref[:, h * D:(h + 1) * D] = (terms[0] + terms[1]) * 0.5


def _combine(acc, den, bst):
    return pl.pallas_call(
        _comb_body,
        grid=(N // NB_ROWS,),
        in_specs=[
            pl.BlockSpec((2, H, NB_ROWS, D), lambda i: (0, 0, i, 0)),
            pl.BlockSpec((1, 2, H, NS, NB_ROWS), lambda i: (i, 0, 0, 0, 0)),
            pl.BlockSpec((2, H, D), lambda i: (0, 0, 0)),
        ],
        out_specs=pl.BlockSpec((NB_ROWS, H * D), lambda i: (i, 0)),
        out_shape=jax.ShapeDtypeStruct((N, H * D), jnp.float32),
    )(acc, den.reshape(2, H, NS, N // NB_ROWS, NB_ROWS).transpose(3, 0, 1, 2, 4),
      bst)


def kernel(X, edge_index_view0, edge_index_view1, W0, al0, ar0, b0, W1, al1, ar1, b1):
    Wst = jnp.stack([W0, W1])
    alst = jnp.stack([al0, al1])
    arst = jnp.stack([ar0, ar1])
    bst = jnp.stack([b0, b1])

    feat, el, er = _prep(X, Wst, alst, arst)
    feat_flat = feat.reshape(2 * H * N, D)
    el_t = el.transpose(0, 2, 1)
    er_t = er.transpose(0, 2, 1)

    ei = jnp.stack([edge_index_view0, edge_index_view1]).astype(jnp.int32)
    pad = jnp.tile(jnp.array([[0], [N]], jnp.int32)[None], (2, 1, E2 - E))
    ei = jnp.concatenate([ei, pad], axis=2)
    edges4 = ei.reshape(2, 2, NS, EC)
    edges6 = ei.reshape(2, 2, NS, NBT, 1, BSZ)

    w, den = _sck1(el_t, er_t, edges4)
    w6 = w.reshape(2, H, NS, NBT, 1, BSZ)
    acc = _sck2(feat_flat, edges6, w6)
    return _combine(acc, den, bst)


# speedup vs baseline: 1.0172x; 1.0172x over previous
import functools

import jax
import jax.numpy as jnp
from jax import lax
from jax.experimental import pallas as pl
from jax.experimental.pallas import tpu as pltpu
from jax.experimental.pallas import tpu_sc as plsc

N = 10000
E = 160000
IN_DIM = 256
H = 4
D = 128
NB_ROWS = 1000
NS = 16
NC = 2
BSZ = 80
NBT = 128
EC = NBT * BSZ
E2 = NS * EC
NSL = 64
SLABS = NBT // NSL
ROWS_PER_TILE = N // NS
ZR = 25


def _prep_body(x_ref, w_ref, al_ref, ar_ref, feat_ref, el_ref, er_ref):
    xb = x_ref[...]
    fb = jnp.dot(xb, w_ref[0], preferred_element_type=jnp.float32)
    f3 = fb.reshape(NB_ROWS, H, D)
    el_ref[0] = (f3 * al_ref[0][None]).sum(-1)
    er_ref[0] = (f3 * ar_ref[0][None]).sum(-1)
    feat_ref[0] = f3.transpose(1, 0, 2)


def _prep(X, Wst, alst, arst):
    return pl.pallas_call(
        _prep_body,
        grid=(2, N // NB_ROWS),
        in_specs=[
            pl.BlockSpec((NB_ROWS, IN_DIM), lambda v, i: (i, 0)),
            pl.BlockSpec((1, IN_DIM, H * D), lambda v, i: (v, 0, 0)),
            pl.BlockSpec((1, H, D), lambda v, i: (v, 0, 0)),
            pl.BlockSpec((1, H, D), lambda v, i: (v, 0, 0)),
        ],
        out_specs=[
            pl.BlockSpec((1, H, NB_ROWS, D), lambda v, i: (v, 0, i, 0)),
            pl.BlockSpec((1, NB_ROWS, H), lambda v, i: (v, i, 0)),
            pl.BlockSpec((1, NB_ROWS, H), lambda v, i: (v, i, 0)),
        ],
        out_shape=[
            jax.ShapeDtypeStruct((2, H, N, D), jnp.float32),
            jax.ShapeDtypeStruct((2, N, H), jnp.float32),
            jax.ShapeDtypeStruct((2, N, H), jnp.float32),
        ],
    )(X, Wst, alst, arst)


def _sck1_body(el_hbm, er_hbm, edges_hbm,
               w_hbm, den_hbm,
               el_v, er_v, src_c, dst_c, w_c, den_v):
    cid = lax.axis_index("c")
    sid = lax.axis_index("s")
    zeros16 = jnp.zeros((16,), jnp.float32)

    for v in range(2):
        for hh in range(2):
            h = cid * 2 + hh

            pltpu.sync_copy(el_hbm.at[v, h], el_v)
            pltpu.sync_copy(er_hbm.at[v, h], er_v.at[pl.ds(0, N)])
            er_v[pl.ds(N, 16)] = zeros16
            pltpu.sync_copy(edges_hbm.at[v, 0, sid], src_c)
            pltpu.sync_copy(edges_hbm.at[v, 1, sid], dst_c)

            def _zd(i, _):
                den_v[pl.ds(i * 16, 16)] = zeros16
                return 0
            lax.fori_loop(0, (N + 16) // 16, _zd, 0)

            def _wk(k, _):
                s16 = src_c[pl.ds(k * 16, 16)]
                d16 = dst_c[pl.ds(k * 16, 16)]
                e16 = (plsc.load_gather(el_v, [s16])
                       + plsc.load_gather(er_v, [d16]))
                e16 = jnp.where(e16 >= 0.0, e16, e16 * 0.2)
                w16 = jnp.exp(e16)
                w_c[pl.ds(k * 16, 16)] = w16
                plsc.addupdate_scatter(den_v, [d16], w16)
                return 0
            lax.fori_loop(0, EC // 16, _wk, 0)

            pltpu.sync_copy(w_c, w_hbm.at[v, h, sid])
            pltpu.sync_copy(den_v.at[pl.ds(0, N)], den_hbm.at[v, h, sid])


def _sck1(el_t, er_t, edges4):
    mesh = plsc.VectorSubcoreMesh(core_axis_name="c", subcore_axis_name="s")
    fn = functools.partial(
        pl.kernel,
        out_type=[
            jax.ShapeDtypeStruct((2, H, NS, EC), jnp.float32),
            jax.ShapeDtypeStruct((2, H, NS, N), jnp.float32),
        ],
        mesh=mesh,
        compiler_params=pltpu.CompilerParams(use_tc_tiling_on_sc=False,
                                             needs_layout_passes=False),
        scratch_types=[
            pltpu.VMEM((N,), jnp.float32),
            pltpu.VMEM((N + 16,), jnp.float32),
            pltpu.VMEM((EC,), jnp.int32),
            pltpu.VMEM((EC,), jnp.int32),
            pltpu.VMEM((EC,), jnp.float32),
            pltpu.VMEM((N + 16,), jnp.float32),
        ],
    )(_sck1_body)
    return fn(el_t, er_t, edges4)


def _sck2_body(feat_hbm, edges_hbm, w_hbm,
               acc_hbm,
               acc_sh,
               gb, srcs, dsts, wsl, gidx, zero_v, gsem):
    cid = lax.axis_index("c")
    sid = lax.axis_index("s")
    row0 = sid * ROWS_PER_TILE
    zeros16 = jnp.zeros((16,), jnp.float32)

    def _zz(i, _):
        for j in range(D // 16):
            zero_v[i, pl.ds(j * 16, 16)] = zeros16
        return 0
    lax.fori_loop(0, ZR, _zz, 0)

    def _build_and_gather(base, g, p):
        def _gi(k, _):
            s16 = srcs[g, pl.ds(k * 16, 16)]
            gidx[p, pl.ds(k * 16, 16)] = s16 + base
            return 0
        lax.fori_loop(0, BSZ // 16, _gi, 0)
        pltpu.async_copy(feat_hbm.at[gidx.at[p]], gb.at[p], gsem.at[p])

    def _wait_gather(p):
        pltpu.make_async_copy(feat_hbm.at[gidx.at[p]], gb.at[p],
                              gsem.at[p]).wait()

    def _scale(g, p):
        def _sk(k, _):
            w16 = wsl[g, pl.ds(k * 16, 16)]
            for i16 in range(16):
                w = w16[i16]
                i = k * 16 + i16
                for j in range(D // 16):
                    gb[p, i, pl.ds(j * 16, 16)] = (
                        gb[p, i, pl.ds(j * 16, 16)] * w)
            return 0
        lax.fori_loop(0, BSZ // 16, _sk, 0)

    for v in range(2):
        for hh in range(2):
            h = cid * 2 + hh
            base = (v * H + h) * N

            def _za(z, _):
                pltpu.sync_copy(zero_v, acc_sh.at[pl.ds(row0 + z * ZR, ZR)])
                return 0
            lax.fori_loop(0, ROWS_PER_TILE // ZR, _za, 0)

            plsc.subcore_barrier()

            def _slab(sl, _):
                pltpu.sync_copy(edges_hbm.at[v, 0, sid, sl], srcs)
                pltpu.sync_copy(edges_hbm.at[v, 1, sid, sl], dsts)
                pltpu.sync_copy(w_hbm.at[v, h, sid, sl], wsl)

                _build_and_gather(base, 0, 0)

                def _pair(t, _):
                    for p in range(2):
                        g = 2 * t + p
                        q = 1 - p
                        _wait_gather(p)
                        @pl.when(g + 1 <= NSL - 1)
                        def _():
                            _build_and_gather(base, g + 1, q)
                        _scale(g, p)
                        pltpu.sync_copy(gb.at[p], acc_sh.at[dsts.at[g]],
                                        add=True)
                    return 0
                lax.fori_loop(0, NSL // 2, _pair, 0)
                return 0
            lax.fori_loop(0, SLABS, _slab, 0)

            plsc.subcore_barrier()

            def _wo(z, _):
                r = row0 + z * ZR
                pltpu.sync_copy(acc_sh.at[pl.ds(r, ZR)],
                                acc_hbm.at[v, h, pl.ds(r, ZR)])
                return 0
            lax.fori_loop(0, ROWS_PER_TILE // ZR, _wo, 0)

            plsc.subcore_barrier()


def _sck2(feat_flat, edges7, w7):
    mesh = plsc.VectorSubcoreMesh(core_axis_name="c", subcore_axis_name="s")
    fn = functools.partial(
        pl.kernel,
        out_type=jax.ShapeDtypeStruct((2, H, N, D), jnp.float32),
        mesh=mesh,
        compiler_params=pltpu.CompilerParams(use_tc_tiling_on_sc=False,
                                             needs_layout_passes=False),
        scratch_types=[
            pltpu.VMEM_SHARED((N + 8, D), jnp.float32),
            pltpu.VMEM((2, BSZ, D), jnp.float32),
            pltpu.VMEM((NSL, BSZ), jnp.int32),
            pltpu.VMEM((NSL, BSZ), jnp.int32),
            pltpu.VMEM((NSL, BSZ), jnp.float32),
            pltpu.VMEM((2, BSZ), jnp.int32),
            pltpu.VMEM((ZR, D), jnp.float32),
            pltpu.SemaphoreType.DMA((2,)),
        ],
    )(_sck2_body)
    return fn(feat_flat, edges7, w7)


def _comb_body(acc_ref, den_ref, b_ref, out_ref):
    dblk = den_ref[0].sum(axis=2)
    for h in range(H):
        terms = []
        for v in range(2):
            numer = acc_ref[v, h]
            dd = dblk[v, h][:, None] + 1e-9
            terms.append(numer / dd + b_ref[v, h][None])
        out_ref[:, h * D:(h + 1) * D] = (terms[0] + terms[1]) * 0.5


def _combine(acc, den, bst):
    return pl.pallas_call(
        _comb_body,
        grid=(N // NB_ROWS,),
        in_specs=[
            pl.BlockSpec((2, H, NB_ROWS, D), lambda i: (0, 0, i, 0)),
            pl.BlockSpec((1, 2, H, NS, NB_ROWS), lambda i: (i, 0, 0, 0, 0)),
            pl.BlockSpec((2, H, D), lambda i: (0, 0, 0)),
        ],
        out_specs=pl.BlockSpec((NB_ROWS, H * D), lambda i: (i, 0)),
        out_shape=jax.ShapeDtypeStruct((N, H * D), jnp.float32),
    )(acc, den.reshape(2, H, NS, N // NB_ROWS, NB_ROWS).transpose(3, 0, 1, 2, 4),
      bst)


def kernel(X, edge_index_view0, edge_index_view1, W0, al0, ar0, b0, W1, al1, ar1, b1):
    Wst = jnp.stack([W0, W1])
    alst = jnp.stack([al0, al1])
    arst = jnp.stack([ar0, ar1])
    bst = jnp.stack([b0, b1])

    feat, el, er = _prep(X, Wst, alst, arst)
    feat_flat = feat.reshape(2 * H * N, D)
    el_t = el.transpose(0, 2, 1)
    er_t = er.transpose(0, 2, 1)

    ei = jnp.stack([edge_index_view0, edge_index_view1]).astype(jnp.int32)
    pad = jnp.tile(jnp.array([[0], [N]], jnp.int32)[None], (2, 1, E2 - E))
    ei = jnp.concatenate([ei, pad], axis=2)
    edges4 = ei.reshape(2, 2, NS, EC)
    edges7 = ei.reshape(2, 2, NS, SLABS, NSL, BSZ)

    w, den = _sck1(el_t, er_t, edges4)
    w7 = w.reshape(2, H, NS, SLABS, NSL, BSZ)
    acc = _sck2(feat_flat, edges7, w7)
    return _combine(acc, den, bst)


# speedup vs baseline: 1.0756x; 1.0574x over previous
import functools

import jax
import jax.numpy as jnp
from jax import lax
from jax.experimental import pallas as pl
from jax.experimental.pallas import tpu as pltpu
from jax.experimental.pallas import tpu_sc as plsc

N = 10000
E = 160000
IN_DIM = 256
H = 4
D = 128
NB_ROWS = 1000
NS = 16
NC = 2
BSZ = 80
NBT = 128
EC = NBT * BSZ
E2 = NS * EC
NSL = 16
SLABS = NBT // NSL
ROWS_PER_TILE = N // NS
ZR = 25


def _prep_body(x_ref, w_ref, al_ref, ar_ref, feat_ref, el_ref, er_ref):
    xb = x_ref[...]
    fb = jnp.dot(xb, w_ref[0], preferred_element_type=jnp.float32)
    f3 = fb.reshape(NB_ROWS, H, D)
    el_ref[0] = (f3 * al_ref[0][None]).sum(-1)
    er_ref[0] = (f3 * ar_ref[0][None]).sum(-1)
    feat_ref[0] = f3.transpose(1, 0, 2)


def _prep(X, Wst, alst, arst):
    return pl.pallas_call(
        _prep_body,
        grid=(2, N // NB_ROWS),
        in_specs=[
            pl.BlockSpec((NB_ROWS, IN_DIM), lambda v, i: (i, 0)),
            pl.BlockSpec((1, IN_DIM, H * D), lambda v, i: (v, 0, 0)),
            pl.BlockSpec((1, H, D), lambda v, i: (v, 0, 0)),
            pl.BlockSpec((1, H, D), lambda v, i: (v, 0, 0)),
        ],
        out_specs=[
            pl.BlockSpec((1, H, NB_ROWS, D), lambda v, i: (v, 0, i, 0)),
            pl.BlockSpec((1, NB_ROWS, H), lambda v, i: (v, i, 0)),
            pl.BlockSpec((1, NB_ROWS, H), lambda v, i: (v, i, 0)),
        ],
        out_shape=[
            jax.ShapeDtypeStruct((2, H, N, D), jnp.float32),
            jax.ShapeDtypeStruct((2, N, H), jnp.float32),
            jax.ShapeDtypeStruct((2, N, H), jnp.float32),
        ],
    )(X, Wst, alst, arst)


def _sck1_body(el_hbm, er_hbm, edges_hbm,
               w_hbm, den_hbm,
               el_v, er_v, src_c, dst_c, w_c, den_v):
    cid = lax.axis_index("c")
    sid = lax.axis_index("s")
    zeros16 = jnp.zeros((16,), jnp.float32)

    for v in range(2):
        for hh in range(2):
            h = cid * 2 + hh

            pltpu.sync_copy(el_hbm.at[v, h], el_v)
            pltpu.sync_copy(er_hbm.at[v, h], er_v.at[pl.ds(0, N)])
            er_v[pl.ds(N, 16)] = zeros16
            pltpu.sync_copy(edges_hbm.at[v, 0, sid], src_c)
            pltpu.sync_copy(edges_hbm.at[v, 1, sid], dst_c)

            def _zd(i, _):
                den_v[pl.ds(i * 16, 16)] = zeros16
                return 0
            lax.fori_loop(0, (N + 16) // 16, _zd, 0)

            def _wk(k, _):
                s16 = src_c[pl.ds(k * 16, 16)]
                d16 = dst_c[pl.ds(k * 16, 16)]
                e16 = (plsc.load_gather(el_v, [s16])
                       + plsc.load_gather(er_v, [d16]))
                e16 = jnp.where(e16 >= 0.0, e16, e16 * 0.2)
                w16 = jnp.exp(e16)
                w_c[pl.ds(k * 16, 16)] = w16
                plsc.addupdate_scatter(den_v, [d16], w16)
                return 0
            lax.fori_loop(0, EC // 16, _wk, 0)

            pltpu.sync_copy(w_c, w_hbm.at[v, h, sid])
            pltpu.sync_copy(den_v.at[pl.ds(0, N)], den_hbm.at[v, h, sid])


def _sck1(el_t, er_t, edges4):
    mesh = plsc.VectorSubcoreMesh(core_axis_name="c", subcore_axis_name="s")
    fn = functools.partial(
        pl.kernel,
        out_type=[
            jax.ShapeDtypeStruct((2, H, NS, EC), jnp.float32),
            jax.ShapeDtypeStruct((2, H, NS, N), jnp.float32),
        ],
        mesh=mesh,
        compiler_params=pltpu.CompilerParams(use_tc_tiling_on_sc=False,
                                             needs_layout_passes=False),
        scratch_types=[
            pltpu.VMEM((N,), jnp.float32),
            pltpu.VMEM((N + 16,), jnp.float32),
            pltpu.VMEM((EC,), jnp.int32),
            pltpu.VMEM((EC,), jnp.int32),
            pltpu.VMEM((EC,), jnp.float32),
            pltpu.VMEM((N + 16,), jnp.float32),
        ],
    )(_sck1_body)
    return fn(el_t, er_t, edges4)


def _sck2_body(feat_hbm, edges_hbm, w_hbm,
               acc_hbm,
               acc_sh,
               gb, srcs, dsts, wsl, gidx, zero_v, gsem, ssem):
    cid = lax.axis_index("c")
    sid = lax.axis_index("s")
    row0 = sid * ROWS_PER_TILE
    zeros16 = jnp.zeros((16,), jnp.float32)

    def _zz(i, _):
        for j in range(D // 16):
            zero_v[i, pl.ds(j * 16, 16)] = zeros16
        return 0
    lax.fori_loop(0, ZR, _zz, 0)

    def _build_and_gather(base, g, p, b):
        def _gi(k, _):
            s16 = srcs[g, pl.ds(k * 16, 16)]
            gidx[p, pl.ds(k * 16, 16)] = s16 + base
            return 0
        lax.fori_loop(0, BSZ // 16, _gi, 0)
        pltpu.async_copy(feat_hbm.at[gidx.at[p]], gb.at[b], gsem.at[b])

    def _wait_gather(p, b):
        pltpu.make_async_copy(feat_hbm.at[gidx.at[p]], gb.at[b],
                              gsem.at[b]).wait()

    def _wait_scatter(p, b):
        pltpu.make_async_copy(gb.at[b], acc_sh.at[dsts.at[0]],
                              ssem.at[p]).wait()

    def _scale(g, b):
        def _sk(k, _):
            w16 = wsl[g, pl.ds(k * 16, 16)]
            for i16 in range(16):
                w = w16[i16]
                i = k * 16 + i16
                for j in range(D // 16):
                    gb[b, i, pl.ds(j * 16, 16)] = (
                        gb[b, i, pl.ds(j * 16, 16)] * w)
            return 0
        lax.fori_loop(0, BSZ // 16, _sk, 0)

    for v in range(2):
        for hh in range(2):
            h = cid * 2 + hh
            base = (v * H + h) * N

            def _za(z, _):
                pltpu.sync_copy(zero_v, acc_sh.at[pl.ds(row0 + z * ZR, ZR)])
                return 0
            lax.fori_loop(0, ROWS_PER_TILE // ZR, _za, 0)

            plsc.subcore_barrier()

            def _slab(sl, _):
                pltpu.sync_copy(edges_hbm.at[v, 0, sid, sl], srcs)
                pltpu.sync_copy(edges_hbm.at[v, 1, sid, sl], dsts)
                pltpu.sync_copy(w_hbm.at[v, h, sid, sl], wsl)

                _build_and_gather(base, 0, 0, 0)
                _build_and_gather(base, 1, 1, 1)

                def _quad(qd, _):
                    for b in range(4):
                        g = 4 * qd + b
                        p = b % 2
                        _wait_gather(p, b)
                        _scale(g, b)
                        @pl.when(g >= 2)
                        def _():
                            _wait_scatter(p, (b + 2) % 4)
                        pltpu.async_copy(gb.at[b], acc_sh.at[dsts.at[g]],
                                        ssem.at[p], add=True)
                        @pl.when(g + 2 <= NSL - 1)
                        def _():
                            _build_and_gather(base, g + 2, p, (b + 2) % 4)
                    return 0
                lax.fori_loop(0, NSL // 4, _quad, 0)

                _wait_scatter(0, (NSL - 2) % 4)
                _wait_scatter(1, (NSL - 1) % 4)
                return 0
            lax.fori_loop(0, SLABS, _slab, 0)

            plsc.subcore_barrier()

            def _wo(z, _):
                r = row0 + z * ZR
                pltpu.sync_copy(acc_sh.at[pl.ds(r, ZR)],
                                acc_hbm.at[v, h, pl.ds(r, ZR)])
                return 0
            lax.fori_loop(0, ROWS_PER_TILE // ZR, _wo, 0)

            plsc.subcore_barrier()


def _sck2(feat_flat, edges7, w7):
    mesh = plsc.VectorSubcoreMesh(core_axis_name="c", subcore_axis_name="s")
    fn = functools.partial(
        pl.kernel,
        out_type=jax.ShapeDtypeStruct((2, H, N, D), jnp.float32),
        mesh=mesh,
        compiler_params=pltpu.CompilerParams(use_tc_tiling_on_sc=False,
                                             needs_layout_passes=False),
        scratch_types=[
            pltpu.VMEM_SHARED((N + 8, D), jnp.float32),
            pltpu.VMEM((4, BSZ, D), jnp.float32),
            pltpu.VMEM((NSL, BSZ), jnp.int32),
            pltpu.VMEM((NSL, BSZ), jnp.int32),
            pltpu.VMEM((NSL, BSZ), jnp.float32),
            pltpu.VMEM((2, BSZ), jnp.int32),
            pltpu.VMEM((ZR, D), jnp.float32),
            pltpu.SemaphoreType.DMA((4,)),
            pltpu.SemaphoreType.DMA((2,)),
        ],
    )(_sck2_body)
    return fn(feat_flat, edges7, w7)


def _comb_body(acc_ref, den_ref, b_ref, out_ref):
    dblk = den_ref[0].sum(axis=2)
    for h in range(H):
        terms = []
        for v in range(2):
            numer = acc_ref[v, h]
            dd = dblk[v, h][:, None] + 1e-9
            terms.append(numer / dd + b_ref[v, h][None])
        out_ref[:, h * D:(h + 1) * D] = (terms[0] + terms[1]) * 0.5


def _combine(acc, den, bst):
    return pl.pallas_call(
        _comb_body,
        grid=(N // NB_ROWS,),
        in_specs=[
            pl.BlockSpec((2, H, NB_ROWS, D), lambda i: (0, 0, i, 0)),
            pl.BlockSpec((1, 2, H, NS, NB_ROWS), lambda i: (i, 0, 0, 0, 0)),
            pl.BlockSpec((2, H, D), lambda i: (0, 0, 0)),
        ],
        out_specs=pl.BlockSpec((NB_ROWS, H * D), lambda i: (i, 0)),
        out_shape=jax.ShapeDtypeStruct((N, H * D), jnp.float32),
    )(acc, den.reshape(2, H, NS, N // NB_ROWS, NB_ROWS).transpose(3, 0, 1, 2, 4),
      bst)


def kernel(X, edge_index_view0, edge_index_view1, W0, al0, ar0, b0, W1, al1, ar1, b1):
    Wst = jnp.stack([W0, W1])
    alst = jnp.stack([al0, al1])
    arst = jnp.stack([ar0, ar1])
    bst = jnp.stack([b0, b1])

    feat, el, er = _prep(X, Wst, alst, arst)
    feat_flat = feat.reshape(2 * H * N, D)
    el_t = el.transpose(0, 2, 1)
    er_t = er.transpose(0, 2, 1)

    ei = jnp.stack([edge_index_view0, edge_index_view1]).astype(jnp.int32)
    pad = jnp.tile(jnp.array([[0], [N]], jnp.int32)[None], (2, 1, E2 - E))
    ei = jnp.concatenate([ei, pad], axis=2)
    edges4 = ei.reshape(2, 2, NS, EC)
    edges7 = ei.reshape(2, 2, NS, SLABS, NSL, BSZ)

    w, den = _sck1(el_t, er_t, edges4)
    w7 = w.reshape(2, H, NS, SLABS, NSL, BSZ)
    acc = _sck2(feat_flat, edges7, w7)
    return _combine(acc, den, bst)


# speedup vs baseline: 1.2816x; 1.1915x over previous
import functools

import jax
import jax.numpy as jnp
from jax import lax
from jax.experimental import pallas as pl
from jax.experimental.pallas import tpu as pltpu
from jax.experimental.pallas import tpu_sc as plsc

N = 10000
E = 160000
IN_DIM = 256
H = 4
D = 128
NB_ROWS = 1000
NS = 16
NC = 2
EC = E // NS
BSZ = 80
SB = 5
NBATCH = EC // BSZ // SB
ROWS_PER_TILE = N // NS
ZR = 25


def _prep_body(x_ref, w_ref, al_ref, ar_ref, feat_ref, el_ref, er_ref):
    xb = x_ref[...]
    fb = jnp.dot(xb, w_ref[0], preferred_element_type=jnp.float32)
    f3 = fb.reshape(NB_ROWS, H, D)
    el_ref[0] = (f3 * al_ref[0][None]).sum(-1)
    er_ref[0] = (f3 * ar_ref[0][None]).sum(-1)
    feat_ref[0] = f3.transpose(1, 0, 2)


def _prep(X, Wst, alst, arst):
    return pl.pallas_call(
        _prep_body,
        grid=(2, N // NB_ROWS),
        in_specs=[
            pl.BlockSpec((NB_ROWS, IN_DIM), lambda v, i: (i, 0)),
            pl.BlockSpec((1, IN_DIM, H * D), lambda v, i: (v, 0, 0)),
            pl.BlockSpec((1, H, D), lambda v, i: (v, 0, 0)),
            pl.BlockSpec((1, H, D), lambda v, i: (v, 0, 0)),
        ],
        out_specs=[
            pl.BlockSpec((1, H, NB_ROWS, D), lambda v, i: (v, 0, i, 0)),
            pl.BlockSpec((1, NB_ROWS, H), lambda v, i: (v, i, 0)),
            pl.BlockSpec((1, NB_ROWS, H), lambda v, i: (v, i, 0)),
        ],
        out_shape=[
            jax.ShapeDtypeStruct((2, H, N, D), jnp.float32),
            jax.ShapeDtypeStruct((2, N, H), jnp.float32),
            jax.ShapeDtypeStruct((2, N, H), jnp.float32),
        ],
    )(X, Wst, alst, arst)


def _sc_body(feat_hbm, el_hbm, er_hbm, edges_hbm,
             acc_hbm, den_hbm,
             acc_sh,
             el_v, er_v, src_v, dst_v, gidx_v, rows_v, w_v, den_v, zero_v):
    cid = lax.axis_index("c")
    sid = lax.axis_index("s")
    row0 = sid * ROWS_PER_TILE

    zeros16 = jnp.zeros((16,), jnp.float32)

    def _zz(i, _):
        for j in range(D // 16):
            zero_v[i, pl.ds(j * 16, 16)] = zeros16
        return 0
    lax.fori_loop(0, ZR, _zz, 0)

    for v in range(2):
        for hh in range(2):
            h = cid * 2 + hh
            base = (v * H + h) * N

            for z in range(ROWS_PER_TILE // ZR):
                pltpu.sync_copy(zero_v, acc_sh.at[pl.ds(row0 + z * ZR, ZR)])

            def _zd(i, _):
                den_v[pl.ds(i * 16, 16)] = zeros16
                return 0
            lax.fori_loop(0, N // 16, _zd, 0)

            pltpu.sync_copy(el_hbm.at[v, h], el_v)
            pltpu.sync_copy(er_hbm.at[v, h], er_v)

            plsc.subcore_barrier()

            def _super(sb, _):
                pltpu.sync_copy(edges_hbm.at[v, 0, sid, sb], src_v)
                pltpu.sync_copy(edges_hbm.at[v, 1, sid, sb], dst_v)

                def _batch(b, _):
                    def _gi(k, _):
                        s16 = src_v[b, pl.ds(k * 16, 16)]
                        gidx_v[0, pl.ds(k * 16, 16)] = s16 + base
                        return 0
                    lax.fori_loop(0, BSZ // 16, _gi, 0)

                    pltpu.sync_copy(feat_hbm.at[gidx_v.at[0]], rows_v)

                    def _wk(k, _):
                        s16 = src_v[b, pl.ds(k * 16, 16)]
                        d16 = dst_v[b, pl.ds(k * 16, 16)]
                        e16 = (plsc.load_gather(el_v, [s16])
                               + plsc.load_gather(er_v, [d16]))
                        e16 = jnp.where(e16 >= 0.0, e16, e16 * 0.2)
                        w16 = jnp.exp(e16)
                        w_v[pl.ds(k * 16, 16)] = w16
                        plsc.addupdate_scatter(den_v, [d16], w16)
                        return 0
                    lax.fori_loop(0, BSZ // 16, _wk, 0)

                    def _sc(k, _):
                        w16 = w_v[pl.ds(k * 16, 16)]
                        for i16 in range(16):
                            w = w16[i16]
                            i = k * 16 + i16
                            for j in range(D // 16):
                                rows_v[i, pl.ds(j * 16, 16)] = (
                                    rows_v[i, pl.ds(j * 16, 16)] * w)
                        return 0
                    lax.fori_loop(0, BSZ // 16, _sc, 0)

                    pltpu.sync_copy(rows_v, acc_sh.at[dst_v.at[b]], add=True)
                    return 0

                lax.fori_loop(0, NBATCH, _batch, 0)
                return 0

            lax.fori_loop(0, SB, _super, 0)

            plsc.subcore_barrier()

            for z in range(ROWS_PER_TILE // ZR):
                r = row0 + z * ZR
                pltpu.sync_copy(acc_sh.at[pl.ds(r, ZR)],
                                acc_hbm.at[v, h, pl.ds(r, ZR)])
            pltpu.sync_copy(den_v, den_hbm.at[v, h, sid])

            plsc.subcore_barrier()


def _sc_edge(feat_flat, el_t, er_t, edges):
    mesh = plsc.VectorSubcoreMesh(core_axis_name="c", subcore_axis_name="s")
    fn = functools.partial(
        pl.kernel,
        out_type=[
            jax.ShapeDtypeStruct((2, H, N, D), jnp.float32),
            jax.ShapeDtypeStruct((2, H, NS, N), jnp.float32),
        ],
        mesh=mesh,
        compiler_params=pltpu.CompilerParams(use_tc_tiling_on_sc=False,
                                             needs_layout_passes=False),
        scratch_types=[
            pltpu.VMEM_SHARED((N, D), jnp.float32),
            pltpu.VMEM((N,), jnp.float32),
            pltpu.VMEM((N,), jnp.float32),
            pltpu.VMEM((NBATCH, BSZ), jnp.int32),
            pltpu.VMEM((NBATCH, BSZ), jnp.int32),
            pltpu.VMEM((1, BSZ), jnp.int32),
            pltpu.VMEM((BSZ, D), jnp.float32),
            pltpu.VMEM((BSZ,), jnp.float32),
            pltpu.VMEM((N,), jnp.float32),
            pltpu.VMEM((ZR, D), jnp.float32),
        ],
    )(_sc_body)
    return fn(feat_flat, el_t, er_t, edges)


def _comb_body(acc_ref, den_ref, b_ref, out_ref):
    dblk = den_ref[0].sum(axis=2)
    for h in range(H):
        terms = []
        for v in range(2):
            numer = acc_ref[v, h]
            dd = dblk[v, h][:, None] + 1e-9
            terms.append(numer / dd + b_ref[v, h][None])
        out_ref[:, h * D:(h + 1) * D] = (terms[0] + terms[1]) * 0.5


def _combine(acc, den, bst):
    return pl.pallas_call(
        _comb_body,
        grid=(N // NB_ROWS,),
        in_specs=[
            pl.BlockSpec((2, H, NB_ROWS, D), lambda i: (0, 0, i, 0)),
            pl.BlockSpec((1, 2, H, NS, NB_ROWS), lambda i: (i, 0, 0, 0, 0)),
            pl.BlockSpec((2, H, D), lambda i: (0, 0, 0)),
        ],
        out_specs=pl.BlockSpec((NB_ROWS, H * D), lambda i: (i, 0)),
        out_shape=jax.ShapeDtypeStruct((N, H * D), jnp.float32),
    )(acc, den.reshape(2, H, NS, N // NB_ROWS, NB_ROWS).transpose(3, 0, 1, 2, 4),
      bst)


def kernel(X, edge_index_view0, edge_index_view1, W0, al0, ar0, b0, W1, al1, ar1, b1):
    Wst = jnp.stack([W0, W1])
    alst = jnp.stack([al0, al1])
    arst = jnp.stack([ar0, ar1])
    bst = jnp.stack([b0, b1])

    feat, el, er = _prep(X, Wst, alst, arst)
    feat_flat = feat.reshape(2 * H * N, D)
    el_t = el.transpose(0, 2, 1)
    er_t = er.transpose(0, 2, 1)

    edges = (jnp.stack([edge_index_view0, edge_index_view1])
             .astype(jnp.int32).reshape(2, 2, NS, SB, NBATCH, BSZ))

    acc, den = _sc_edge(feat_flat, el_t, er_t, edges)
    return _combine(acc, den, bst)
